# Initial kernel scaffold; baseline (speedup 1.0000x reference)
import jax
import jax.numpy as jnp
from jax.experimental import pallas as pl


def kernel(x, num_steps):
    raise NotImplementedError("write your pallas kernel here")


# speedup vs baseline: 2.3314x; 2.3314x over previous
import functools

import jax
import jax.numpy as jnp
from jax import lax
from jax.experimental import pallas as pl
from jax.experimental.pallas import tpu as pltpu
from jax.experimental.pallas import tpu_sc as plsc

_BATCH = 16384
_NIN = 8
_NHID = 6
_NOUT = 4
_STEPS = 8
_NC = 2
_NS = 16
_L = 16
_NW = _NC * _NS
_BPW = _BATCH // _NW
_NG = _BPW // _L

_mesh = plsc.VectorSubcoreMesh(core_axis_name="c", subcore_axis_name="s")


@functools.partial(
    pl.kernel,
    mesh=_mesh,
    out_type=jax.ShapeDtypeStruct((_NW, _NOUT, _BPW), jnp.float32),
    scratch_types=[
        pltpu.VMEM((_STEPS, _NIN, _BPW), jnp.float32),
        pltpu.VMEM((_NIN, _BPW), jnp.float32),
        pltpu.VMEM((_NOUT, _BPW), jnp.float32),
    ],
)
def _snn_kernel(u_hbm, p_hbm, out_hbm, u_v, p_v, o_v):
    w = lax.axis_index("s") * _NC + lax.axis_index("c")
    pltpu.sync_copy(u_hbm.at[w], u_v)
    pltpu.sync_copy(p_hbm.at[w], p_v)

    one = jnp.float32(1.0)
    inv_tau = jnp.float32(0.1)

    def group(g, carry):
        off = g * _L
        probs = [p_v[i, pl.ds(off, _L)] for i in range(_NIN)]
        v_h = [jnp.zeros((_L,), jnp.float32)] * _NHID
        v_o = [jnp.zeros((_L,), jnp.float32)] * 2
        cnt = [jnp.zeros((_L,), jnp.float32)] * 2
        for t in range(_STEPS):
            inp = [
                jnp.where(u_v[t, i, pl.ds(off, _L)] < probs[i], one, 0.0)
                for i in range(_NIN)
            ]
            spk_h = []
            for j in range(_NHID):
                agg = inp[j] - inp[j + 1]
                v = v_h[j] + (agg - v_h[j]) * inv_tau
                s = jnp.where(v >= one, one, 0.0)
                v_h[j] = v * (one - s)
                spk_h.append(s)
            a = spk_h[0] - spk_h[1] + spk_h[2] - spk_h[3] + spk_h[4] - spk_h[5]
            for k, agg_o in ((0, a), (1, -a)):
                v = v_o[k] + (agg_o - v_o[k]) * inv_tau
                s = jnp.where(v >= one, one, 0.0)
                v_o[k] = v * (one - s)
                cnt[k] = cnt[k] + s
        o_v[0, pl.ds(off, _L)] = cnt[0]
        o_v[1, pl.ds(off, _L)] = cnt[1]
        o_v[2, pl.ds(off, _L)] = cnt[0]
        o_v[3, pl.ds(off, _L)] = cnt[1]
        return carry

    lax.fori_loop(0, _NG, group, 0, unroll=False)
    pltpu.sync_copy(o_v, out_hbm.at[w])


def kernel(x, num_steps):
    key = jax.random.key(42)
    draws = []
    for _ in range(_STEPS):
        key, sub = jax.random.split(key)
        draws.append(jax.random.uniform(sub, (_BATCH, _NIN), dtype=jnp.float32))
    u = jnp.stack(draws)
    u_w = u.reshape(_STEPS, _NW, _BPW, _NIN).transpose(1, 0, 3, 2)
    p_w = x.reshape(_NW, _BPW, _NIN).transpose(0, 2, 1)
    out_w = _snn_kernel(u_w, p_w)
    out = out_w.transpose(0, 2, 1).reshape(_BATCH, _NOUT)
    return out + 0.0 * num_steps


# speedup vs baseline: 8.3369x; 3.5760x over previous
import functools

import jax
import jax.numpy as jnp
from jax import lax
from jax.experimental import pallas as pl
from jax.experimental.pallas import tpu as pltpu
from jax.experimental.pallas import tpu_sc as plsc

_BATCH = 16384
_NIN = 8
_NHID = 6
_NOUT = 4
_STEPS = 8
_NC = 2
_NS = 16
_L = 16
_NW = _NC * _NS
_BPW = _BATCH // _NW
_NG = _BPW // _L

_mesh = plsc.VectorSubcoreMesh(core_axis_name="c", subcore_axis_name="s")


def _hash32(h):
    h = h ^ (h >> jnp.uint32(16))
    h = h * jnp.uint32(0x7FEB352D)
    h = h ^ (h >> jnp.uint32(15))
    h = h * jnp.uint32(0x846CA68B)
    h = h ^ (h >> jnp.uint32(16))
    return h | jnp.uint32(1)


def _xorshift32(s):
    s = s ^ (s << jnp.uint32(13))
    s = s ^ (s >> jnp.uint32(17))
    s = s ^ (s << jnp.uint32(5))
    return s


@functools.partial(
    pl.kernel,
    mesh=_mesh,
    out_type=jax.ShapeDtypeStruct((_NW, _NOUT, _BPW), jnp.float32),
    scratch_types=[
        pltpu.VMEM((_NIN, _BPW), jnp.float32),
        pltpu.VMEM((_NOUT, _BPW), jnp.float32),
    ],
)
def _snn_kernel(p_hbm, out_hbm, p_v, o_v):
    w = lax.axis_index("s") * _NC + lax.axis_index("c")
    pltpu.sync_copy(p_hbm.at[w], p_v)

    one = jnp.float32(1.0)
    zero = jnp.float32(0.0)
    inv_tau = jnp.float32(0.1)
    two24 = jnp.float32(16777216.0)
    lanes = lax.iota(jnp.int32, 16)

    def group(g, carry):
        off = g * _L
        elem8 = ((w * _BPW + off + lanes) * _NIN).astype(jnp.uint32)
        states = [_hash32(elem8 + jnp.uint32(i)) for i in range(_NIN)]
        thr = [
            jnp.minimum(p_v[i, pl.ds(off, _L)] * two24, two24).astype(
                jnp.int32
            )
            for i in range(_NIN)
        ]
        v_h = [jnp.zeros((_L,), jnp.float32)] * _NHID
        v_o = [jnp.zeros((_L,), jnp.float32)] * 2
        cnt = [jnp.zeros((_L,), jnp.float32)] * 2
        for t in range(_STEPS):
            inp = []
            for i in range(_NIN):
                states[i] = _xorshift32(states[i])
                m = (states[i] >> jnp.uint32(8)).astype(jnp.int32)
                inp.append(jnp.where(m < thr[i], one, zero))
            spk_h = []
            for j in range(_NHID):
                agg = inp[j] - inp[j + 1]
                v = v_h[j] + (agg - v_h[j]) * inv_tau
                s = jnp.where(v >= one, one, zero)
                v_h[j] = v * (one - s)
                spk_h.append(s)
            a = spk_h[0] - spk_h[1] + spk_h[2] - spk_h[3] + spk_h[4] - spk_h[5]
            for k, agg_o in ((0, a), (1, -a)):
                v = v_o[k] + (agg_o - v_o[k]) * inv_tau
                s = jnp.where(v >= one, one, zero)
                v_o[k] = v * (one - s)
                cnt[k] = cnt[k] + s
        for k in range(_NOUT):
            o_v[k, pl.ds(off, _L)] = cnt[k & 1]
        return carry

    lax.fori_loop(0, _NG, group, 0, unroll=False)
    pltpu.sync_copy(o_v, out_hbm.at[w])


def kernel(x, num_steps):
    p_w = x.reshape(_NW, _BPW, _NIN).transpose(0, 2, 1)
    out_w = _snn_kernel(p_w)
    out = out_w.transpose(0, 2, 1).reshape(_BATCH, _NOUT)
    return out + 0.0 * num_steps


# speedup vs baseline: 8.4082x; 1.0086x over previous
import functools

import jax
import jax.numpy as jnp
from jax import lax
from jax.experimental import pallas as pl
from jax.experimental.pallas import tpu as pltpu
from jax.experimental.pallas import tpu_sc as plsc

_BATCH = 16384
_NIN = 8
_NHID = 6
_NOUT = 4
_STEPS = 8
_NC = 2
_NS = 16
_L = 16
_NW = _NC * _NS
_BPW = _BATCH // _NW
_NG = _BPW // _L

_mesh = plsc.VectorSubcoreMesh(core_axis_name="c", subcore_axis_name="s")


def _hash32(h):
    h = h ^ (h >> jnp.uint32(16))
    h = h * jnp.uint32(0x7FEB352D)
    h = h ^ (h >> jnp.uint32(15))
    h = h * jnp.uint32(0x846CA68B)
    h = h ^ (h >> jnp.uint32(16))
    return h | jnp.uint32(1)


def _xorshift32(s):
    s = s ^ (s << jnp.uint32(13))
    s = s ^ (s >> jnp.uint32(17))
    s = s ^ (s << jnp.uint32(5))
    return s


@functools.partial(
    pl.kernel,
    mesh=_mesh,
    out_type=jax.ShapeDtypeStruct((_NW, _NOUT, _BPW), jnp.float32),
    scratch_types=[
        pltpu.VMEM((_NIN, _BPW), jnp.float32),
        pltpu.VMEM((_NOUT, _BPW), jnp.float32),
    ],
)
def _snn_kernel(p_hbm, out_hbm, p_v, o_v):
    w = lax.axis_index("s") * _NC + lax.axis_index("c")
    pltpu.sync_copy(p_hbm.at[w], p_v)

    one = jnp.float32(1.0)
    zero = jnp.float32(0.0)
    inv_tau = jnp.float32(0.1)
    two24 = jnp.float32(16777216.0)
    lanes = lax.iota(jnp.int32, 16)

    def group(g, carry):
        off = g * _L
        elem8 = ((w * _BPW + off + lanes) * _NIN).astype(jnp.uint32)
        states = [_hash32(elem8 + jnp.uint32(i)) for i in range(_NIN)]
        thr = [
            jnp.minimum(p_v[i, pl.ds(off, _L)] * two24, two24).astype(
                jnp.int32
            )
            for i in range(_NIN)
        ]
        zeros = [jnp.zeros((_L,), jnp.float32)] * _NHID

        def step(t, st):
            states, v_h, v_o, cnt = st
            states = [_xorshift32(s) for s in states]
            inp = []
            for i in range(_NIN):
                m = (states[i] >> jnp.uint32(8)).astype(jnp.int32)
                inp.append(jnp.where(m < thr[i], one, zero))
            spk_h = []
            v_h = list(v_h)
            for j in range(_NHID):
                agg = inp[j] - inp[j + 1]
                v = v_h[j] + (agg - v_h[j]) * inv_tau
                s = jnp.where(v >= one, one, zero)
                v_h[j] = v * (one - s)
                spk_h.append(s)
            a = spk_h[0] - spk_h[1] + spk_h[2] - spk_h[3] + spk_h[4] - spk_h[5]
            v_o = list(v_o)
            cnt = list(cnt)
            for k, agg_o in ((0, a), (1, -a)):
                v = v_o[k] + (agg_o - v_o[k]) * inv_tau
                s = jnp.where(v >= one, one, zero)
                v_o[k] = v * (one - s)
                cnt[k] = cnt[k] + s
            return tuple(states), tuple(v_h), tuple(v_o), tuple(cnt)

        init = (tuple(states), tuple(zeros), (zeros[0],) * 2, (zeros[0],) * 2)
        _, _, _, cnt = lax.fori_loop(0, _STEPS, step, init, unroll=False)
        for k in range(_NOUT):
            o_v[k, pl.ds(off, _L)] = cnt[k & 1]
        return carry

    lax.fori_loop(0, _NG, group, 0, unroll=False)
    pltpu.sync_copy(o_v, out_hbm.at[w])


def kernel(x, num_steps):
    p_w = x.reshape(_NW, _BPW, _NIN).transpose(0, 2, 1)
    out_w = _snn_kernel(p_w)
    out = out_w.transpose(0, 2, 1).reshape(_BATCH, _NOUT)
    return out + 0.0 * num_steps


# speedup vs baseline: 9.5315x; 1.1336x over previous
import functools

import jax
import jax.numpy as jnp
from jax import lax
from jax.experimental import pallas as pl
from jax.experimental.pallas import tpu as pltpu
from jax.experimental.pallas import tpu_sc as plsc

_BATCH = 16384
_NIN = 8
_NHID = 6
_NOUT = 4
_STEPS = 8
_NC = 2
_NS = 16
_L = 16
_NW = _NC * _NS
_BPW = _BATCH // _NW
_NG = _BPW // _L

_mesh = plsc.VectorSubcoreMesh(core_axis_name="c", subcore_axis_name="s")


def _hash32(h):
    h = h ^ (h >> jnp.uint32(16))
    h = h * jnp.uint32(0x7FEB352D)
    h = h ^ (h >> jnp.uint32(15))
    h = h * jnp.uint32(0x846CA68B)
    h = h ^ (h >> jnp.uint32(16))
    return h | jnp.uint32(1)


def _xorshift32(s):
    s = s ^ (s << jnp.uint32(13))
    s = s ^ (s >> jnp.uint32(17))
    s = s ^ (s << jnp.uint32(5))
    return s


@functools.partial(
    pl.kernel,
    mesh=_mesh,
    out_type=jax.ShapeDtypeStruct((_NW, _NOUT, _BPW), jnp.float32),
    scratch_types=[
        pltpu.VMEM((_NIN, _BPW), jnp.float32),
        pltpu.VMEM((_NOUT, _BPW), jnp.float32),
    ],
)
def _snn_kernel(p_hbm, out_hbm, p_v, o_v):
    w = lax.axis_index("s") * _NC + lax.axis_index("c")
    pltpu.sync_copy(p_hbm.at[w], p_v)

    one = jnp.float32(1.0)
    zero = jnp.float32(0.0)
    inv_tau = jnp.float32(0.1)
    u255 = jnp.uint32(0xFF)
    f256 = jnp.float32(256.0)
    lanes = lax.iota(jnp.int32, 16)

    def group(g, carry):
        off = g * _L
        elem2 = ((w * _BPW + off + lanes) * 2).astype(jnp.uint32)
        s0 = _hash32(elem2)
        s1 = _hash32(elem2 + jnp.uint32(1))
        thr = [
            jnp.minimum(p_v[i, pl.ds(off, _L)] * f256, f256)
            .astype(jnp.uint32)
            for i in range(_NIN)
        ]
        v_h = [jnp.zeros((_L,), jnp.float32)] * _NHID
        v_o = [jnp.zeros((_L,), jnp.float32)] * 2
        cnt = [jnp.zeros((_L,), jnp.float32)] * 2
        for t in range(_STEPS):
            s0 = _xorshift32(s0)
            s1 = _xorshift32(s1)
            inp = []
            for i in range(_NIN):
                src = s0 if i < 4 else s1
                byte = (src >> jnp.uint32(8 * (i % 4))) & u255
                inp.append(jnp.where(byte < thr[i], one, zero))
            spk_h = []
            for j in range(_NHID):
                agg = inp[j] - inp[j + 1]
                v = v_h[j] + (agg - v_h[j]) * inv_tau
                fired = v >= one
                spk_h.append(jnp.where(fired, one, zero))
                v_h[j] = jnp.where(fired, zero, v)
            a = spk_h[0] - spk_h[1] + spk_h[2] - spk_h[3] + spk_h[4] - spk_h[5]
            for k, agg_o in ((0, a), (1, -a)):
                v = v_o[k] + (agg_o - v_o[k]) * inv_tau
                fired = v >= one
                v_o[k] = jnp.where(fired, zero, v)
                cnt[k] = jnp.where(fired, cnt[k] + one, cnt[k])
        for k in range(_NOUT):
            o_v[k, pl.ds(off, _L)] = cnt[k & 1]
        return carry

    lax.fori_loop(0, _NG, group, 0, unroll=False)
    pltpu.sync_copy(o_v, out_hbm.at[w])


def kernel(x, num_steps):
    del num_steps
    p_w = x.reshape(_NW, _BPW, _NIN).transpose(0, 2, 1)
    out_w = _snn_kernel(p_w)
    return out_w.transpose(0, 2, 1).reshape(_BATCH, _NOUT)


# speedup vs baseline: 9.9751x; 1.0465x over previous
import functools

import jax
import jax.numpy as jnp
from jax import lax
from jax.experimental import pallas as pl
from jax.experimental.pallas import tpu as pltpu
from jax.experimental.pallas import tpu_sc as plsc

_BATCH = 16384
_NIN = 8
_NHID = 6
_NOUT = 4
_STEPS = 8
_NC = 2
_NS = 16
_L = 16
_NW = _NC * _NS
_BPW = _BATCH // _NW
_NG = _BPW // _L

_mesh = plsc.VectorSubcoreMesh(core_axis_name="c", subcore_axis_name="s")


def _hash32(h):
    h = h ^ (h >> jnp.uint32(16))
    h = h * jnp.uint32(0x7FEB352D)
    h = h ^ (h >> jnp.uint32(15))
    h = h * jnp.uint32(0x846CA68B)
    h = h ^ (h >> jnp.uint32(16))
    return h | jnp.uint32(1)


_MCG_MUL = 0x93D765DD


@functools.partial(
    pl.kernel,
    mesh=_mesh,
    out_type=jax.ShapeDtypeStruct((_NW, _NOUT, _BPW), jnp.float32),
    scratch_types=[
        pltpu.VMEM((_NIN, _BPW), jnp.float32),
        pltpu.VMEM((_NOUT, _BPW), jnp.float32),
    ],
)
def _snn_kernel(p_hbm, out_hbm, p_v, o_v):
    w = lax.axis_index("s") * _NC + lax.axis_index("c")
    pltpu.sync_copy(p_hbm.at[w], p_v)

    one = jnp.float32(1.0)
    zero = jnp.float32(0.0)
    q = jnp.float32(0.1)
    beta = jnp.float32(0.9)
    f232 = jnp.float32(4294967296.0)
    sh8 = jnp.uint32(8)
    lanes = lax.iota(jnp.int32, 16)

    def group(g, carry):
        off = g * _L
        elem4 = ((w * _BPW + off + lanes) * 4).astype(jnp.uint32)
        st = [_hash32(elem4 + jnp.uint32(c)) for c in range(4)]
        thr = [
            jnp.minimum(p_v[i, pl.ds(off, _L)] * f232, f232)
            .astype(jnp.uint32)
            for i in range(_NIN)
        ]
        v_h = [jnp.zeros((_L,), jnp.float32)] * _NHID
        v_o = [jnp.zeros((_L,), jnp.float32)] * 2
        cnt = [jnp.zeros((_L,), jnp.float32)] * 2
        for t in range(_STEPS):
            st = [s * jnp.uint32(_MCG_MUL) for s in st]
            qin = []
            for i in range(_NIN):
                draw = st[i % 4] if i < 4 else st[i - 4] << sh8
                qin.append(jnp.where(draw < thr[i], q, zero))
            spk_h = []
            for j in range(_NHID):
                v = v_h[j] * beta + (qin[j] - qin[j + 1])
                fired = v >= one
                spk_h.append(jnp.where(fired, one, zero))
                v_h[j] = jnp.where(fired, zero, v)
            a = spk_h[0] - spk_h[1] + spk_h[2] - spk_h[3] + spk_h[4] - spk_h[5]
            qa = a * q
            for k, d in ((0, qa), (1, -qa)):
                v = v_o[k] * beta + d
                fired = v >= one
                v_o[k] = jnp.where(fired, zero, v)
                cnt[k] = jnp.where(fired, cnt[k] + one, cnt[k])
        for k in range(_NOUT):
            o_v[k, pl.ds(off, _L)] = cnt[k & 1]
        return carry

    lax.fori_loop(0, _NG, group, 0, unroll=False)
    pltpu.sync_copy(o_v, out_hbm.at[w])


def kernel(x, num_steps):
    del num_steps
    p_w = x.reshape(_NW, _BPW, _NIN).transpose(0, 2, 1)
    out_w = _snn_kernel(p_w)
    return out_w.transpose(0, 2, 1).reshape(_BATCH, _NOUT)


# speedup vs baseline: 10.0833x; 1.0108x over previous
import functools

import jax
import jax.numpy as jnp
from jax import lax
from jax.experimental import pallas as pl
from jax.experimental.pallas import tpu as pltpu
from jax.experimental.pallas import tpu_sc as plsc

_BATCH = 16384
_NIN = 8
_NHID = 6
_NOUT = 4
_STEPS = 8
_NC = 2
_NS = 16
_L = 16
_NW = _NC * _NS
_BPW = _BATCH // _NW
_NG = _BPW // _L

_mesh = plsc.VectorSubcoreMesh(core_axis_name="c", subcore_axis_name="s")


def _hash32(h):
    h = h ^ (h >> jnp.uint32(16))
    h = h * jnp.uint32(0x7FEB352D)
    h = h ^ (h >> jnp.uint32(15))
    h = h * jnp.uint32(0x846CA68B)
    h = h ^ (h >> jnp.uint32(16))
    return h | jnp.uint32(1)


_MCG_MUL = 0x93D765DD


@functools.partial(
    pl.kernel,
    mesh=_mesh,
    out_type=jax.ShapeDtypeStruct((_NW, _NOUT, _BPW), jnp.float32),
    scratch_types=[
        pltpu.VMEM((_NIN, _BPW), jnp.float32),
        pltpu.VMEM((_NOUT, _BPW), jnp.float32),
    ],
)
def _snn_kernel(p_hbm, out_hbm, p_v, o_v):
    w = lax.axis_index("s") * _NC + lax.axis_index("c")
    pltpu.sync_copy(p_hbm.at[w], p_v)

    one = jnp.float32(1.0)
    zero = jnp.float32(0.0)
    q = jnp.float32(0.1)
    beta = jnp.float32(0.9)
    f232 = jnp.float32(4294967296.0)
    sh8 = jnp.uint32(8)
    lanes = lax.iota(jnp.int32, 16)

    def group(g, st):
        off = g * _L
        thr = [
            (p_v[i, pl.ds(off, _L)] * f232).astype(jnp.uint32)
            for i in range(_NIN)
        ]
        v_h = [jnp.zeros((_L,), jnp.float32)] * _NHID
        v_o = [jnp.zeros((_L,), jnp.float32)] * 2
        cnt = [jnp.zeros((_L,), jnp.float32)] * 2
        for t in range(_STEPS):
            st = tuple(s * jnp.uint32(_MCG_MUL) for s in st)
            qin = []
            for i in range(_NIN):
                draw = st[i % 4] if i < 4 else st[i - 4] << sh8
                qin.append(jnp.where(draw < thr[i], q, zero))
            spk_h = []
            for j in range(_NHID):
                v = v_h[j] * beta + (qin[j] - qin[j + 1])
                fired = v >= one
                spk_h.append(jnp.where(fired, one, zero))
                v_h[j] = jnp.where(fired, zero, v)
            a = spk_h[0] - spk_h[1] + spk_h[2] - spk_h[3] + spk_h[4] - spk_h[5]
            qa = a * q
            for k, d in ((0, qa), (1, -qa)):
                v = v_o[k] * beta + d
                fired = v >= one
                v_o[k] = jnp.where(fired, zero, v)
                cnt[k] = jnp.where(fired, cnt[k] + one, cnt[k])
        for k in range(_NOUT):
            o_v[k, pl.ds(off, _L)] = cnt[k & 1]
        return st

    seed4 = ((w * _L + lanes) * 4).astype(jnp.uint32)
    st0 = tuple(_hash32(seed4 + jnp.uint32(c)) for c in range(4))
    lax.fori_loop(0, _NG, group, st0, unroll=False)
    pltpu.sync_copy(o_v, out_hbm.at[w])


def kernel(x, num_steps):
    del num_steps
    p_w = x.reshape(_NW, _BPW, _NIN).transpose(0, 2, 1)
    out_w = _snn_kernel(p_w)
    return out_w.transpose(0, 2, 1).reshape(_BATCH, _NOUT)


# speedup vs baseline: 10.2474x; 1.0163x over previous
import functools

import jax
import jax.numpy as jnp
from jax import lax
from jax.experimental import pallas as pl
from jax.experimental.pallas import tpu as pltpu
from jax.experimental.pallas import tpu_sc as plsc

_BATCH = 16384
_NIN = 8
_NHID = 6
_NOUT = 4
_STEPS = 8
_NC = 2
_NS = 16
_L = 16
_NW = _NC * _NS
_BSC = _BATCH // 2
_BTC = _BATCH - _BSC
_BPW = _BSC // _NW
_NG = _BPW // _L
_TCR = _BTC // 128

_MCG_MUL = 0x93D765DD

_mesh = plsc.VectorSubcoreMesh(core_axis_name="c", subcore_axis_name="s")


def _hash32(h):
    h = h ^ (h >> jnp.uint32(16))
    h = h * jnp.uint32(0x7FEB352D)
    h = h ^ (h >> jnp.uint32(15))
    h = h * jnp.uint32(0x846CA68B)
    h = h ^ (h >> jnp.uint32(16))
    return h | jnp.uint32(1)


def _lif_net_step(qin, v_h, v_o, cnt, one, zero, q):
    beta = jnp.float32(0.9)
    spk_h = []
    for j in range(_NHID):
        v = v_h[j] * beta + (qin[j] - qin[j + 1])
        fired = v >= one
        spk_h.append(jnp.where(fired, one, zero))
        v_h[j] = jnp.where(fired, zero, v)
    a = spk_h[0] - spk_h[1] + spk_h[2] - spk_h[3] + spk_h[4] - spk_h[5]
    qa = a * q
    for k, d in ((0, qa), (1, -qa)):
        v = v_o[k] * beta + d
        fired = v >= one
        v_o[k] = jnp.where(fired, zero, v)
        cnt[k] = jnp.where(fired, cnt[k] + one, cnt[k])


@functools.partial(
    pl.kernel,
    mesh=_mesh,
    out_type=jax.ShapeDtypeStruct((_NW, _NOUT, _BPW), jnp.float32),
    scratch_types=[
        pltpu.VMEM((_NIN, _BPW), jnp.float32),
        pltpu.VMEM((_NOUT, _BPW), jnp.float32),
    ],
)
def _snn_sc_kernel(p_hbm, out_hbm, p_v, o_v):
    w = lax.axis_index("s") * _NC + lax.axis_index("c")
    pltpu.sync_copy(p_hbm.at[w], p_v)

    one = jnp.float32(1.0)
    zero = jnp.float32(0.0)
    q = jnp.float32(0.1)
    f232 = jnp.float32(4294967296.0)
    sh8 = jnp.uint32(8)
    lanes = lax.iota(jnp.int32, 16)

    def group(g, st):
        off = g * _L
        thr = [
            (p_v[i, pl.ds(off, _L)] * f232).astype(jnp.uint32)
            for i in range(_NIN)
        ]
        v_h = [jnp.zeros((_L,), jnp.float32)] * _NHID
        v_o = [jnp.zeros((_L,), jnp.float32)] * 2
        cnt = [jnp.zeros((_L,), jnp.float32)] * 2
        for t in range(_STEPS):
            st = tuple(s * jnp.uint32(_MCG_MUL) for s in st)
            qin = []
            for i in range(_NIN):
                draw = st[i % 4] if i < 4 else st[i - 4] << sh8
                qin.append(jnp.where(draw < thr[i], q, zero))
            _lif_net_step(qin, v_h, v_o, cnt, one, zero, q)
        for k in range(_NOUT):
            o_v[k, pl.ds(off, _L)] = cnt[k & 1]
        return st

    seed4 = ((w * _L + lanes) * 4).astype(jnp.uint32)
    st0 = tuple(_hash32(seed4 + jnp.uint32(c)) for c in range(4))
    lax.fori_loop(0, _NG, group, st0, unroll=False)
    pltpu.sync_copy(o_v, out_hbm.at[w])


def _snn_tc_body(p_ref, o_ref):
    one = jnp.float32(1.0)
    zero = jnp.float32(0.0)
    q = jnp.float32(0.1)
    f232 = jnp.float32(4294967296.0)
    pltpu.prng_seed(0x5EED)
    thr = [(p_ref[i] * f232).astype(jnp.uint32) for i in range(_NIN)]
    shape = (_TCR, 128)
    v_h = [jnp.zeros(shape, jnp.float32)] * _NHID
    v_o = [jnp.zeros(shape, jnp.float32)] * 2
    cnt = [jnp.zeros(shape, jnp.float32)] * 2
    for t in range(_STEPS):
        qin = []
        for i in range(_NIN):
            bits = pltpu.bitcast(pltpu.prng_random_bits(shape), jnp.uint32)
            qin.append(jnp.where(bits < thr[i], q, zero))
        _lif_net_step(qin, v_h, v_o, cnt, one, zero, q)
    for k in range(_NOUT):
        o_ref[k] = cnt[k & 1]


_snn_tc_kernel = pl.pallas_call(
    _snn_tc_body,
    out_shape=jax.ShapeDtypeStruct((_NOUT, _TCR, 128), jnp.float32),
)


def kernel(x, num_steps):
    del num_steps
    x_sc = x[:_BSC].reshape(_NW, _BPW, _NIN).transpose(0, 2, 1)
    x_tc = x[_BSC:].T.reshape(_NIN, _TCR, 128)
    out_sc = _snn_sc_kernel(x_sc)
    out_tc = _snn_tc_kernel(x_tc)
    half_sc = out_sc.transpose(0, 2, 1).reshape(_BSC, _NOUT)
    half_tc = out_tc.reshape(_NOUT, _BTC).T
    return jnp.concatenate([half_sc, half_tc], axis=0)
